# baseline (speedup 1.0000x reference)
import jax
import jax.numpy as jnp
from jax.experimental import pallas as pl

_BLK = 256


def _body(pay_ref, pert_ref, emb_ref, out_ref):
    j = pl.program_id(1)

    @pl.when(j == 0)
    def _():
        mask = (pay_ref[...] == 1).astype(out_ref.dtype)
        out_ref[...] = emb_ref[...] + (pert_ref[...] * mask)[None]

    @pl.when(j != 0)
    def _():
        out_ref[...] = emb_ref[...]


def kernel(embedded_input, watermark_payload, perturbation_vectors):
    b, s, d = embedded_input.shape
    l = perturbation_vectors.shape[0]
    pay2d = watermark_payload.reshape(l, 1)
    return pl.pallas_call(
        _body,
        grid=(b, s // _BLK),
        in_specs=[
            pl.BlockSpec((l, 1), lambda bi, j: (0, 0)),
            pl.BlockSpec((l, d), lambda bi, j: (0, 0)),
            pl.BlockSpec((1, _BLK, d), lambda bi, j: (bi, j, 0)),
        ],
        out_specs=pl.BlockSpec((1, _BLK, d), lambda bi, j: (bi, j, 0)),
        out_shape=jax.ShapeDtypeStruct((b, s, d), embedded_input.dtype),
    )(pay2d, perturbation_vectors, embedded_input)


# speedup vs baseline: 1.1087x; 1.1087x over previous
import jax
import jax.numpy as jnp
from jax.experimental import pallas as pl

_BLK = 1024
_L = 256


def _body(pay_ref, pert_ref, emb_ref, out_ref):
    j = pl.program_id(1)

    @pl.when(j == 0)
    def _():
        mask = (pay_ref[...] == 1).astype(out_ref.dtype)
        out_ref[0, :_L, :] = emb_ref[0, :_L, :] + pert_ref[...] * mask
        out_ref[0, _L:, :] = emb_ref[0, _L:, :]

    @pl.when(j != 0)
    def _():
        out_ref[...] = emb_ref[...]


def kernel(embedded_input, watermark_payload, perturbation_vectors):
    b, s, d = embedded_input.shape
    l = perturbation_vectors.shape[0]
    pay2d = watermark_payload.reshape(l, 1)
    return pl.pallas_call(
        _body,
        grid=(b, s // _BLK),
        in_specs=[
            pl.BlockSpec((l, 1), lambda bi, j: (0, 0)),
            pl.BlockSpec((l, d), lambda bi, j: (0, 0)),
            pl.BlockSpec((1, _BLK, d), lambda bi, j: (bi, j, 0)),
        ],
        out_specs=pl.BlockSpec((1, _BLK, d), lambda bi, j: (bi, j, 0)),
        out_shape=jax.ShapeDtypeStruct((b, s, d), embedded_input.dtype),
    )(pay2d, perturbation_vectors, embedded_input)
